# baseline (speedup 1.0000x reference)
import functools

import jax
import jax.numpy as jnp
from jax import lax
from jax.experimental import pallas as pl
from jax.experimental.pallas import tpu as pltpu
from jax.experimental.pallas import tpu_sc as plsc

VOCAB = 100000
DM = 512
B = 1024
T = 200

NC = 2
NS = 16
NW = NC * NS
EPW = B // NW
CHS = (72, 64, 64)
OFFS = (0, 72, 136)
NCH = len(CHS)
CHMAX = max(CHS)
LANES = 16
KCH = DM // LANES


def _pool_body(
    tok_hbm, table_hbm, out_hbm, idx_v, bufs, accb, s0, s1, s2, o0, o1
):
    sems = (s0, s1, s2)
    osems = (o0, o1)
    wid = lax.axis_index("s") * NC + lax.axis_index("c")
    tbase = pl.multiple_of(wid * (EPW * T), 8)
    pltpu.sync_copy(tok_hbm.at[pl.ds(tbase, EPW * T)], idx_v)

    def fire(e, c):
        off = pl.multiple_of(e * T + OFFS[c], 8)
        idx_slice = idx_v.at[pl.ds(off, CHS[c])]
        dst = bufs.at[c, pl.ds(0, CHS[c])]
        return pltpu.async_copy(table_hbm.at[idx_slice], dst, sems[c])

    def accum_chunk(c, accs):
        def row_body(r, a):
            return tuple(
                a[k] + bufs[c, r, pl.ds(k * LANES, LANES)] for k in range(KCH)
            )

        return lax.fori_loop(0, CHS[c], row_body, accs)

    UNROLL = 8

    def block_body(pblk, carry):
        e0 = pblk * UNROLL
        handles = [fire(e0, c) for c in range(NCH)]
        houts = [None] * UNROLL
        for u in range(UNROLL):
            accs = tuple(jnp.zeros((LANES,), jnp.float32) for _ in range(KCH))
            nxt = [None] * NCH
            for c in range(NCH):
                handles[c].wait()
                accs = accum_chunk(c, accs)
                if u + 1 < UNROLL:
                    nxt[c] = fire(e0 + u + 1, c)
            slot = u % 2
            if u >= 2:
                houts[u - 2].wait()
            for k in range(KCH):
                accb[slot, pl.ds(k * LANES, LANES)] = accs[k]
            houts[u] = pltpu.async_copy(
                accb.at[slot], out_hbm.at[wid * EPW + e0 + u], osems[slot]
            )
            handles = nxt
        houts[UNROLL - 2].wait()
        houts[UNROLL - 1].wait()
        return carry

    lax.fori_loop(0, EPW // UNROLL, block_body, 0)


_pool = functools.partial(
    pl.kernel,
    out_type=jax.ShapeDtypeStruct((B, DM), jnp.float32),
    mesh=plsc.VectorSubcoreMesh(core_axis_name="c", subcore_axis_name="s"),
    scratch_types=[
        pltpu.VMEM((EPW * T,), jnp.int32),
        pltpu.VMEM((NCH, CHMAX, DM), jnp.float32),
        pltpu.VMEM((2, DM), jnp.float32),
        pltpu.SemaphoreType.DMA,
        pltpu.SemaphoreType.DMA,
        pltpu.SemaphoreType.DMA,
        pltpu.SemaphoreType.DMA,
        pltpu.SemaphoreType.DMA,
    ],
)(_pool_body)


def _linear_body(x_ref, w_ref, b_ref, o_ref):
    x = x_ref[...] * (1.0 / T)
    o_ref[...] = (
        lax.dot_general(
            x,
            w_ref[...],
            dimension_numbers=(((1,), (1,)), ((), ())),
            preferred_element_type=jnp.float32,
        )
        + b_ref[...]
    )


def kernel(token_ids, emb_table, W, b):
    tok = token_ids.reshape(B * T).astype(jnp.int32)
    pooled = _pool(tok, emb_table)
    out = pl.pallas_call(
        _linear_body,
        out_shape=jax.ShapeDtypeStruct((B, DM), jnp.float32),
    )(pooled, W, b.reshape(1, DM))
    return out


# speedup vs baseline: 1.0153x; 1.0153x over previous
import functools

import jax
import jax.numpy as jnp
from jax import lax
from jax.experimental import pallas as pl
from jax.experimental.pallas import tpu as pltpu
from jax.experimental.pallas import tpu_sc as plsc

VOCAB = 100000
DM = 512
B = 1024
T = 200

NC = 2
NS = 16
NW = NC * NS
EPW = B // NW
CHS = (40, 40, 40, 40, 40)
OFFS = (0, 40, 80, 120, 160)
NCH = len(CHS)
CHMAX = max(CHS)
LANES = 16
KCH = DM // LANES


def _pool_body(
    tok_hbm, table_hbm, out_hbm, idx_v, bufs, accb, s0, s1, s2, s3, s4, o0, o1
):
    sems = (s0, s1, s2, s3, s4)
    osems = (o0, o1)
    wid = lax.axis_index("s") * NC + lax.axis_index("c")
    tbase = pl.multiple_of(wid * (EPW * T), 8)
    pltpu.sync_copy(tok_hbm.at[pl.ds(tbase, EPW * T)], idx_v)

    def fire(e, c):
        off = pl.multiple_of(e * T + OFFS[c], 8)
        idx_slice = idx_v.at[pl.ds(off, CHS[c])]
        dst = bufs.at[c, pl.ds(0, CHS[c])]
        return pltpu.async_copy(table_hbm.at[idx_slice], dst, sems[c])

    def accum_chunk(c, accs):
        def row_body(r, a):
            return tuple(
                a[k] + bufs[c, r, pl.ds(k * LANES, LANES)] for k in range(KCH)
            )

        return lax.fori_loop(0, CHS[c], row_body, accs)

    UNROLL = 8

    def block_body(pblk, carry):
        e0 = pblk * UNROLL
        handles = [fire(e0, c) for c in range(NCH)]
        houts = [None] * UNROLL
        for u in range(UNROLL):
            accs = tuple(jnp.zeros((LANES,), jnp.float32) for _ in range(KCH))
            nxt = [None] * NCH
            for c in range(NCH):
                handles[c].wait()
                accs = accum_chunk(c, accs)
                if u + 1 < UNROLL:
                    nxt[c] = fire(e0 + u + 1, c)
            slot = u % 2
            if u >= 2:
                houts[u - 2].wait()
            for k in range(KCH):
                accb[slot, pl.ds(k * LANES, LANES)] = accs[k]
            houts[u] = pltpu.async_copy(
                accb.at[slot], out_hbm.at[wid * EPW + e0 + u], osems[slot]
            )
            handles = nxt
        houts[UNROLL - 2].wait()
        houts[UNROLL - 1].wait()
        return carry

    lax.fori_loop(0, EPW // UNROLL, block_body, 0)


_pool = functools.partial(
    pl.kernel,
    out_type=jax.ShapeDtypeStruct((B, DM), jnp.float32),
    mesh=plsc.VectorSubcoreMesh(core_axis_name="c", subcore_axis_name="s"),
    scratch_types=[
        pltpu.VMEM((EPW * T,), jnp.int32),
        pltpu.VMEM((NCH, CHMAX, DM), jnp.float32),
        pltpu.VMEM((2, DM), jnp.float32),
        pltpu.SemaphoreType.DMA,
        pltpu.SemaphoreType.DMA,
        pltpu.SemaphoreType.DMA,
        pltpu.SemaphoreType.DMA,
        pltpu.SemaphoreType.DMA,
        pltpu.SemaphoreType.DMA,
        pltpu.SemaphoreType.DMA,
    ],
)(_pool_body)


def _linear_body(x_ref, w_ref, b_ref, o_ref):
    x = x_ref[...] * (1.0 / T)
    o_ref[...] = (
        lax.dot_general(
            x,
            w_ref[...],
            dimension_numbers=(((1,), (1,)), ((), ())),
            preferred_element_type=jnp.float32,
        )
        + b_ref[...]
    )


def kernel(token_ids, emb_table, W, b):
    tok = token_ids.reshape(B * T).astype(jnp.int32)
    pooled = _pool(tok, emb_table)
    out = pl.pallas_call(
        _linear_body,
        out_shape=jax.ShapeDtypeStruct((B, DM), jnp.float32),
    )(pooled, W, b.reshape(1, DM))
    return out
